# Initial kernel scaffold; baseline (speedup 1.0000x reference)
import jax
import jax.numpy as jnp
from jax.experimental import pallas as pl


def kernel(x, edge_index, batch, W1, b1, W2, b2, W3, b3, g1, be1, g2, be2, g3, be3, O1, ob1, O2, ob2, O3, ob3):
    raise NotImplementedError("write your pallas kernel here")


# speedup vs baseline: 10.5171x; 10.5171x over previous
import functools
import math

import jax
import jax.numpy as jnp
from jax import lax
from jax.experimental import pallas as pl
from jax.experimental.pallas import tpu as pltpu
from jax.experimental.pallas import tpu_sc as plsc

N = 10000
E = 320000
D = 128
G = 64
EPS = 1e-5
BN_SCALE = 1.0 / math.sqrt(1.0 + EPS)

NC = 2
NS = 16
NW = NC * NS

NPAD = 10240
TILE_ROWS = NPAD // NS
EC = 128
CHUNKS = (E + NW * EC - 1) // (NW * EC)
EPAD = NW * CHUNKS * EC

RBLK = 512
NBLK = NPAD // RBLK

_sc_mesh = plsc.VectorSubcoreMesh(core_axis_name="c", subcore_axis_name="s",
                                  num_cores=NC, num_subcores=NS)


@functools.partial(
    pl.kernel,
    out_type=jax.ShapeDtypeStruct((NC, NPAD, D), jnp.float32),
    mesh=_sc_mesh,
    scratch_types=[
        pltpu.VMEM((CHUNKS, EC), jnp.int32),
        pltpu.VMEM((EC, D), jnp.float32),
        pltpu.VMEM_SHARED((NPAD, D), jnp.float32),
    ],
)
def _deg_sc(dst_hbm, zrows_hbm, ones_hbm, out_hbm, dst_v, ones_v, acc_sh):
    c = lax.axis_index("c")
    s = lax.axis_index("s")
    wid = c * NS + s
    base = s * TILE_ROWS
    pltpu.sync_copy(zrows_hbm, acc_sh.at[pl.ds(base, TILE_ROWS)])
    pltpu.sync_copy(dst_hbm.at[wid], dst_v)
    pltpu.sync_copy(ones_hbm, ones_v)
    plsc.subcore_barrier()

    def body(j, carry):
        pltpu.sync_copy(ones_v, acc_sh.at[dst_v.at[j]], add=True)
        return carry

    lax.fori_loop(0, CHUNKS, body, 0)
    plsc.subcore_barrier()
    pltpu.sync_copy(acc_sh.at[pl.ds(base, TILE_ROWS)],
                    out_hbm.at[c, pl.ds(base, TILE_ROWS)])


@functools.partial(
    pl.kernel,
    out_type=jax.ShapeDtypeStruct((NC, NPAD, D), jnp.float32),
    mesh=_sc_mesh,
    scratch_types=[
        pltpu.VMEM((CHUNKS, EC), jnp.int32),
        pltpu.VMEM((CHUNKS, EC), jnp.int32),
        pltpu.VMEM((EC, D), jnp.float32),
        pltpu.VMEM_SHARED((NPAD, D), jnp.float32),
        pltpu.SemaphoreType.DMA,
    ],
)
def _edge_sc(y_hbm, src_hbm, dst_hbm, zrows_hbm, out_hbm,
             src_v, dst_v, rows_v, acc_sh, sem):
    c = lax.axis_index("c")
    s = lax.axis_index("s")
    wid = c * NS + s
    base = s * TILE_ROWS
    pltpu.sync_copy(zrows_hbm, acc_sh.at[pl.ds(base, TILE_ROWS)])
    pltpu.sync_copy(src_hbm.at[wid], src_v)
    pltpu.sync_copy(dst_hbm.at[wid], dst_v)
    plsc.subcore_barrier()

    def body(j, carry):
        pltpu.async_copy(y_hbm.at[src_v.at[j]], rows_v, sem).wait()
        pltpu.sync_copy(rows_v, acc_sh.at[dst_v.at[j]], add=True)
        return carry

    lax.fori_loop(0, CHUNKS, body, 0)
    plsc.subcore_barrier()
    pltpu.sync_copy(acc_sh.at[pl.ds(base, TILE_ROWS)],
                    out_hbm.at[c, pl.ds(base, TILE_ROWS)])


def _full(shape):
    return pl.BlockSpec(shape, lambda i: tuple(0 for _ in shape))


def _tc_first_body(deg_ref, x_ref, w_ref, y_ref, dis_ref):
    deg = deg_ref[...]
    d = deg[0, :, 0:1] + deg[1, :, 0:1] + 1.0
    dis = lax.rsqrt(d)
    dis_ref[...] = dis
    y_ref[...] = dis * jnp.dot(x_ref[...], w_ref[...],
                               preferred_element_type=jnp.float32)


def _tc_first(degp, xpad, W):
    return pl.pallas_call(
        _tc_first_body,
        grid=(NBLK,),
        in_specs=[
            pl.BlockSpec((NC, RBLK, D), lambda i: (0, i, 0)),
            pl.BlockSpec((RBLK, D), lambda i: (i, 0)),
            _full((D, D)),
        ],
        out_specs=[
            pl.BlockSpec((RBLK, D), lambda i: (i, 0)),
            pl.BlockSpec((RBLK, 1), lambda i: (i, 0)),
        ],
        out_shape=[
            jax.ShapeDtypeStruct((NPAD, D), jnp.float32),
            jax.ShapeDtypeStruct((NPAD, 1), jnp.float32),
        ],
    )(degp, xpad, W)


def _tc_mid_body(z_ref, y_ref, dis_ref, g_ref, b_ref, be_ref, w_ref, *rest):
    if len(rest) == 3:
        prev_ref, h_ref, y2_ref = rest
    else:
        prev_ref = None
        h_ref, y2_ref = rest
    z = z_ref[...]
    conv = dis_ref[...] * (z[0] + z[1] + y_ref[...]) + b_ref[...]
    h = jax.nn.relu(g_ref[...] * BN_SCALE * conv + be_ref[...])
    if prev_ref is not None:
        h = h + prev_ref[...]
    h_ref[...] = h
    y2_ref[...] = dis_ref[...] * jnp.dot(h, w_ref[...],
                                         preferred_element_type=jnp.float32)


def _tc_mid(zp, y, dis, g, b, be, Wnext, prev=None):
    ins = [zp, y, dis, g.reshape(1, D), b.reshape(1, D), be.reshape(1, D),
           Wnext]
    specs = [
        pl.BlockSpec((NC, RBLK, D), lambda i: (0, i, 0)),
        pl.BlockSpec((RBLK, D), lambda i: (i, 0)),
        pl.BlockSpec((RBLK, 1), lambda i: (i, 0)),
        _full((1, D)), _full((1, D)), _full((1, D)),
        _full((D, D)),
    ]
    if prev is not None:
        ins.append(prev)
        specs.append(pl.BlockSpec((RBLK, D), lambda i: (i, 0)))
    return pl.pallas_call(
        _tc_mid_body,
        grid=(NBLK,),
        in_specs=specs,
        out_specs=[
            pl.BlockSpec((RBLK, D), lambda i: (i, 0)),
            pl.BlockSpec((RBLK, D), lambda i: (i, 0)),
        ],
        out_shape=[
            jax.ShapeDtypeStruct((NPAD, D), jnp.float32),
            jax.ShapeDtypeStruct((NPAD, D), jnp.float32),
        ],
    )(*ins)


def _tc_final_body(z_ref, y_ref, dis_ref, g_ref, b_ref, be_ref, prev_ref,
                   batch_ref, o1_ref, ob1_ref, o2_ref, ob2_ref, o3_ref,
                   ob3_ref, out_ref, sums_sc, cnt_sc):
    i = pl.program_id(0)

    @pl.when(i == 0)
    def _():
        sums_sc[...] = jnp.zeros_like(sums_sc)
        cnt_sc[...] = jnp.zeros_like(cnt_sc)

    z = z_ref[...]
    conv = dis_ref[...] * (z[0] + z[1] + y_ref[...]) + b_ref[...]
    h = jax.nn.relu(g_ref[...] * BN_SCALE * conv + be_ref[...])
    h = h + prev_ref[...]

    ids = batch_ref[0, 0, :]
    gids = lax.broadcasted_iota(jnp.int32, (G, RBLK), 0)
    onehot = (ids[None, :] == gids).astype(jnp.float32)
    sums_sc[...] += jnp.dot(onehot, h, preferred_element_type=jnp.float32)
    cnt_sc[...] += jnp.sum(onehot, axis=1, keepdims=True)

    @pl.when(i == NBLK - 1)
    def _():
        hg = sums_sc[...] / jnp.maximum(cnt_sc[...], 1.0)
        o = jnp.tanh(jnp.dot(hg, o1_ref[...],
                             preferred_element_type=jnp.float32) + ob1_ref[...])
        o = jnp.tanh(jnp.dot(o, o2_ref[...],
                             preferred_element_type=jnp.float32) + ob2_ref[...])
        out_ref[...] = jnp.dot(o, o3_ref[...],
                               preferred_element_type=jnp.float32) + ob3_ref[...]


def _tc_final(zp, y, dis, g, b, be, prev, batchp,
              O1, ob1, O2, ob2, O3, ob3):
    return pl.pallas_call(
        _tc_final_body,
        grid=(NBLK,),
        in_specs=[
            pl.BlockSpec((NC, RBLK, D), lambda i: (0, i, 0)),
            pl.BlockSpec((RBLK, D), lambda i: (i, 0)),
            pl.BlockSpec((RBLK, 1), lambda i: (i, 0)),
            _full((1, D)), _full((1, D)), _full((1, D)),
            pl.BlockSpec((RBLK, D), lambda i: (i, 0)),
            pl.BlockSpec((1, 1, RBLK), lambda i: (i, 0, 0)),
            _full((D, D)), _full((1, D)),
            _full((D, D)), _full((1, D)),
            _full((D, 1)), _full((1, 1)),
        ],
        out_specs=pl.BlockSpec((G, 1), lambda i: (0, 0)),
        out_shape=jax.ShapeDtypeStruct((G, 1), jnp.float32),
        scratch_shapes=[
            pltpu.VMEM((G, D), jnp.float32),
            pltpu.VMEM((G, 1), jnp.float32),
        ],
    )(zp, y, dis, g.reshape(1, D), b.reshape(1, D), be.reshape(1, D), prev,
      batchp, O1, ob1.reshape(1, D), O2, ob2.reshape(1, D), O3,
      ob3.reshape(1, 1))


def kernel(x, edge_index, batch, W1, b1, W2, b2, W3, b3,
           g1, be1, g2, be2, g3, be3, O1, ob1, O2, ob2, O3, ob3):
    src = edge_index[0]
    dst = edge_index[1]
    pad_e = EPAD - E
    srcp = jnp.concatenate(
        [src, jnp.zeros((pad_e,), jnp.int32)]).reshape(NW, CHUNKS, EC)
    dstp = jnp.concatenate(
        [dst, jnp.full((pad_e,), NPAD - 1, jnp.int32)]).reshape(NW, CHUNKS, EC)
    xpad = jnp.concatenate([x, jnp.zeros((NPAD - N, D), jnp.float32)])
    batchp = jnp.concatenate(
        [batch, jnp.full((NPAD - N,), G, jnp.int32)]).reshape(NBLK, 1, RBLK)
    zrows = jnp.zeros((TILE_ROWS, D), jnp.float32)
    ones_rows = jnp.ones((EC, D), jnp.float32)

    degp = _deg_sc(dstp, zrows, ones_rows)
    y1, dis = _tc_first(degp, xpad, W1)
    z1 = _edge_sc(y1, srcp, dstp, zrows)
    h1, y2 = _tc_mid(z1, y1, dis, g1, b1, be1, W2)
    z2 = _edge_sc(y2, srcp, dstp, zrows)
    h2, y3 = _tc_mid(z2, y2, dis, g2, b2, be2, W3, prev=h1)
    z3 = _edge_sc(y3, srcp, dstp, zrows)
    out = _tc_final(z3, y3, dis, g3, b3, be3, h2, batchp,
                    O1, ob1, O2, ob2, O3, ob3)
    return out.reshape(G)
